# baseline (speedup 1.0000x reference)
import functools

import jax
import jax.numpy as jnp
from jax import lax
from jax.experimental import pallas as pl
from jax.experimental.pallas import tpu as pltpu
from jax.experimental.pallas import tpu_sc as plsc

D = 16
B = 4096
NK = 50
NC = 2
NS = 16
L = 16
NW = NC * NS
BPW = B // NW
G = BPW // L

_LN2 = 0.6931471805599453


def _rsqrt(y):
    i = plsc.bitcast(y, jnp.int32)
    i = 0x5F3759DF - (i >> 1)
    r = plsc.bitcast(i, jnp.float32)
    r = r * (1.5 - 0.5 * y * r * r)
    r = r * (1.5 - 0.5 * y * r * r)
    return r


def _log(x):
    i = plsc.bitcast(x, jnp.int32)
    k = ((i >> 23) - 127).astype(jnp.float32)
    m = plsc.bitcast((i & 0x007FFFFF) | 0x3F800000, jnp.float32)
    z = (m - 1.0) / (m + 1.0)
    z2 = z * z
    p = 2.0 * z * (1.0 + z2 * (1.0 / 3.0 + z2 * (1.0 / 5.0 + z2 * (1.0 / 7.0 + z2 * (1.0 / 9.0)))))
    return k * _LN2 + p


def _sc_lorentz(table, i_arr, ksw):
    mesh = plsc.VectorSubcoreMesh(core_axis_name="c", subcore_axis_name="s")

    @functools.partial(
        pl.kernel,
        out_type=jax.ShapeDtypeStruct((B,), jnp.float32),
        mesh=mesh,
        compiler_params=pltpu.CompilerParams(
            needs_layout_passes=False, use_tc_tiling_on_sc=False),
        scratch_types=[
            pltpu.VMEM((NK, BPW), jnp.int32),
            pltpu.VMEM((BPW,), jnp.int32),
            pltpu.VMEM((BPW, D), jnp.float32),
            pltpu.VMEM((NK * BPW, D), jnp.float32),
            pltpu.VMEM((BPW,), jnp.float32),
            pltpu.SemaphoreType.DMA,
            pltpu.SemaphoreType.DMA,
        ],
    )
    def k(table_hbm, i_hbm, ksw_hbm, out_hbm,
          ks_idx, i_idx, ui_rows, uk_rows, loss_v, sem_ui, sem_uk):
        wid = lax.axis_index("s") * NC + lax.axis_index("c")
        base = wid * BPW

        pltpu.sync_copy(i_hbm.at[pl.ds(base, BPW)], i_idx)
        pltpu.sync_copy(ksw_hbm.at[wid], ks_idx)

        ui_cp = pltpu.async_copy(table_hbm.at[i_idx], ui_rows, sem_ui)
        uk_cps = []
        for c in range(NK):
            uk_cps.append(pltpu.async_copy(
                table_hbm.at[ks_idx.at[c]],
                uk_rows.at[pl.ds(c * BPW, BPW)], sem_uk))
        ui_cp.wait()
        for cp in uk_cps:
            cp.wait()

        iota = lax.iota(jnp.int32, L)
        dsplat = [jnp.full((L,), d, jnp.int32) for d in range(D)]

        for g in range(G):
            gidx = iota + (g * L)
            uiT = []
            for d in range(D):
                v = plsc.load_gather(ui_rows, [gidx, dsplat[d]])
                uiT.append(-v if d == 0 else v)

            def pair_e(n):
                ridx = iota + (n * BPW + g * L)
                acc = uiT[0] * plsc.load_gather(uk_rows, [ridx, dsplat[0]])
                for d in range(1, D):
                    acc = acc + uiT[d] * plsc.load_gather(uk_rows, [ridx, dsplat[d]])
                dd = jnp.maximum(-acc, 1.0)
                y = (dd - 1.0) * (dd + 1.0)
                return dd - y * _rsqrt(y)

            e0 = pair_e(0)

            def body(n, accS):
                return accS + pair_e(n)

            accS = lax.fori_loop(1, NK, body, e0)
            lg = _log((accS + 1e-6) / e0)
            loss_v[pl.ds(g * L, L)] = lg

        pltpu.sync_copy(loss_v, out_hbm.at[pl.ds(base, BPW)])

    return k(table, i_arr, ksw)


def kernel(table, I, Ks):
    i_arr = I.astype(jnp.int32)
    ks32 = Ks.astype(jnp.int32)
    ksw = ks32.T.reshape(NK, NW, BPW).transpose(1, 0, 2)
    return _sc_lorentz(table.astype(jnp.float32), i_arr, ksw)
